# baseline (speedup 1.0000x reference)
import jax
import jax.numpy as jnp
from jax.experimental import pallas as pl
from jax.experimental.pallas import tpu as pltpu

S_BLK = 2048


def _add_pe_kernel(x_ref, pe_ref, o_ref):
    o_ref[...] = x_ref[...] + pe_ref[...][None]


def kernel(input_seq, pe):
    B, S, D = input_seq.shape
    D_BLK = 512
    grid = (D // D_BLK, B)
    return pl.pallas_call(
        _add_pe_kernel,
        grid=grid,
        in_specs=[
            pl.BlockSpec((1, S, D_BLK), lambda i, b: (b, 0, i)),
            pl.BlockSpec((S, D_BLK), lambda i, b: (0, i)),
        ],
        out_specs=pl.BlockSpec((1, S, D_BLK), lambda i, b: (b, 0, i)),
        out_shape=jax.ShapeDtypeStruct((B, S, D), input_seq.dtype),
        compiler_params=pltpu.CompilerParams(
            dimension_semantics=("parallel", "parallel"),
        ),
    )(input_seq, pe)
